# baseline (speedup 1.0000x reference)
import functools

import jax
import jax.numpy as jnp
from jax import lax
from jax.experimental import pallas as pl
from jax.experimental.pallas import tpu as pltpu
from jax.experimental.pallas import tpu_sc as plsc

B = 16384
EMB = 64
TDIM = 10

_info = plsc.get_sparse_core_info()
_NC, _NS = _info.num_cores, _info.num_subcores
_NW = _NC * _NS
_BPW = B // _NW


@functools.partial(
    pl.kernel,
    mesh=plsc.VectorSubcoreMesh(core_axis_name="c", subcore_axis_name="s"),
    compiler_params=pltpu.CompilerParams(use_tc_tiling_on_sc=False),
    out_type=[
        jax.ShapeDtypeStruct((B, EMB), jnp.float32),
        jax.ShapeDtypeStruct((B, EMB), jnp.float32),
    ],
    scratch_types=[
        pltpu.VMEM((_BPW,), jnp.int32),
        pltpu.VMEM((_BPW,), jnp.int32),
        pltpu.VMEM((_BPW, EMB), jnp.float32),
        pltpu.VMEM((_BPW, EMB), jnp.float32),
        pltpu.SemaphoreType.DMA,
    ],
)
def _sc_gather(rest_table, menu_table, ridx, midx,
               rest_out, menu_out,
               ridx_v, midx_v, rbuf, mbuf, sem):
    wid = lax.axis_index("s") * _NC + lax.axis_index("c")
    base = wid * _BPW

    pltpu.sync_copy(ridx.at[pl.ds(base, _BPW)], ridx_v)
    pltpu.sync_copy(midx.at[pl.ds(base, _BPW)], midx_v)
    rcp = pltpu.async_copy(rest_table.at[ridx_v], rbuf, sem)
    mcp = pltpu.async_copy(menu_table.at[midx_v], mbuf, sem)
    rcp.wait()
    pltpu.sync_copy(rbuf, rest_out.at[pl.ds(base, _BPW)])
    mcp.wait()
    pltpu.sync_copy(mbuf, menu_out.at[pl.ds(base, _BPW)])


def _dense_body(rest_ref, menu_ref, temp_ref, rb_ref, mb_ref, gb_ref,
                tW1_ref, tb1_ref, tW2_ref, tb2_ref, tW3_ref, tb3_ref,
                iW1r_ref, iW1m_ref, iW1t_ref, ib1_ref,
                iW2_ref, ib2_ref, iW3_ref, ib3_ref, out_ref):
    f32 = jnp.float32
    cdim = (((1,), (0,)), ((), ()))
    rest = rest_ref[...]
    menu = menu_ref[...]
    temp = temp_ref[...]
    mf = jnp.sum(rest * menu, axis=1, keepdims=True)
    h = jnp.maximum(
        lax.dot_general(temp, tW1_ref[...], cdim, preferred_element_type=f32)
        + tb1_ref[...], 0.0)
    h = jnp.maximum(
        lax.dot_general(h, tW2_ref[...], cdim, preferred_element_type=f32)
        + tb2_ref[...], 0.0)
    t_score = (lax.dot_general(h, tW3_ref[...], cdim, preferred_element_type=f32)
               + tb3_ref[...])
    g = (lax.dot_general(rest, iW1r_ref[...], cdim, preferred_element_type=f32)
         + lax.dot_general(menu, iW1m_ref[...], cdim, preferred_element_type=f32)
         + lax.dot_general(temp, iW1t_ref[...], cdim, preferred_element_type=f32)
         + ib1_ref[...])
    g = jnp.maximum(g, 0.0)
    g = jnp.maximum(
        lax.dot_general(g, iW2_ref[...], cdim, preferred_element_type=f32)
        + ib2_ref[...], 0.0)
    i_score = (lax.dot_general(g, iW3_ref[...], cdim, preferred_element_type=f32)
               + ib3_ref[...])
    pred = gb_ref[...] + rb_ref[...] + mb_ref[...] + mf + t_score + i_score
    out_ref[...] = jax.nn.sigmoid(pred[:, 0])


def kernel(restaurant_idx, menu_idx, temporal_features, rest_table, menu_table,
           rest_bias_table, menu_bias_table, global_bias, tW1, tb1, tW2, tb2,
           tW3, tb3, iW1, ib1, iW2, ib2, iW3, ib3):
    ridx = restaurant_idx.astype(jnp.int32)
    midx = menu_idx.astype(jnp.int32)
    rest_emb, menu_emb = _sc_gather(rest_table, menu_table, ridx, midx)
    rest_b = jnp.take(rest_bias_table.reshape(-1), ridx).reshape(B, 1)
    menu_b = jnp.take(menu_bias_table.reshape(-1), midx).reshape(B, 1)

    BS = 2048
    grid = (B // BS,)
    row = lambda c: pl.BlockSpec((BS, c), lambda i: (i, 0))
    full = lambda shape: pl.BlockSpec(shape, lambda i: (0, 0))
    out = pl.pallas_call(
        _dense_body,
        grid=grid,
        in_specs=[
            row(EMB), row(EMB), row(TDIM), row(1), row(1),
            full((1, 1)),
            full((TDIM, 32)), full((1, 32)),
            full((32, 16)), full((1, 16)),
            full((16, 1)), full((1, 1)),
            full((EMB, 128)), full((EMB, 128)), full((TDIM, 128)), full((1, 128)),
            full((128, 64)), full((1, 64)),
            full((64, 1)), full((1, 1)),
        ],
        out_specs=pl.BlockSpec((BS,), lambda i: (i,)),
        out_shape=jax.ShapeDtypeStruct((B,), jnp.float32),
    )(
        rest_emb, menu_emb, temporal_features, rest_b, menu_b,
        global_bias.reshape(1, 1),
        tW1, tb1.reshape(1, 32), tW2, tb2.reshape(1, 16),
        tW3, tb3.reshape(1, 1),
        iW1[:EMB], iW1[EMB:2 * EMB], iW1[2 * EMB:], ib1.reshape(1, 128),
        iW2, ib2.reshape(1, 64), iW3, ib3.reshape(1, 1),
    )
    return out
